# Initial kernel scaffold; baseline (speedup 1.0000x reference)
import jax
import jax.numpy as jnp
from jax.experimental import pallas as pl


def kernel(x, edge_index, edge_attr, M, b_edge, W_root, bias):
    raise NotImplementedError("write your pallas kernel here")


# speedup vs baseline: 2.7134x; 2.7134x over previous
import functools

import jax
import jax.numpy as jnp
from jax import lax
from jax.experimental import pallas as pl
from jax.experimental.pallas import tpu as pltpu
from jax.experimental.pallas import tpu_sc as plsc

N = 10000
E = 160000
IN_C = 128
OUT_C = 64
D_EDGE = 4

NC, NS = 2, 16
NW = NC * NS
CHUNK = 128
NCHUNKS = E // CHUNK
TRIPS = (NCHUNKS + NW - 1) // NW
N_PAD = 10240
STRIPE = N_PAD // NS


def _gather_body(x_hbm, src_hbm, out_hbm, idx_v, rows_v, sem):
    wid = lax.axis_index("s") * NC + lax.axis_index("c")

    def trip(j, _):
        cid = wid + j * NW

        @pl.when(cid < NCHUNKS)
        def _():
            base = cid * CHUNK
            pltpu.sync_copy(src_hbm.at[pl.ds(base, CHUNK)], idx_v)
            pltpu.async_copy(x_hbm.at[idx_v], rows_v, sem).wait()
            pltpu.sync_copy(rows_v, out_hbm.at[pl.ds(base, CHUNK)])

        return _

    lax.fori_loop(0, TRIPS, trip, None)


def _sc_gather(x, src):
    mesh = plsc.VectorSubcoreMesh(core_axis_name="c", subcore_axis_name="s")
    return pl.kernel(
        _gather_body,
        out_type=jax.ShapeDtypeStruct((E, IN_C), jnp.float32),
        mesh=mesh,
        scratch_types=[
            pltpu.VMEM((CHUNK,), jnp.int32),
            pltpu.VMEM((CHUNK, IN_C), jnp.float32),
            pltpu.SemaphoreType.DMA,
        ],
    )(x, src)


def _scatter_body(msg_hbm, dst_hbm, zeros_hbm, out_hbm, idx_v, rows_v, acc_sh, sem):
    c = lax.axis_index("c")
    s = lax.axis_index("s")
    wid = s * NC + c

    pltpu.sync_copy(zeros_hbm.at[pl.ds(s * STRIPE, STRIPE)],
                    acc_sh.at[pl.ds(s * STRIPE, STRIPE)])
    plsc.subcore_barrier()

    def trip(j, _):
        cid = wid + j * NW

        @pl.when(cid < NCHUNKS)
        def _():
            base = cid * CHUNK
            pltpu.sync_copy(dst_hbm.at[pl.ds(base, CHUNK)], idx_v)
            pltpu.sync_copy(msg_hbm.at[pl.ds(base, CHUNK)], rows_v)
            pltpu.sync_copy(rows_v, acc_sh.at[idx_v], add=True)

        return _

    lax.fori_loop(0, TRIPS, trip, None)
    plsc.subcore_barrier()

    pltpu.sync_copy(acc_sh.at[pl.ds(s * STRIPE, STRIPE)],
                    out_hbm.at[c, pl.ds(s * STRIPE, STRIPE)])


def _sc_scatter(msg2, dst, zeros_hbm):
    mesh = plsc.VectorSubcoreMesh(core_axis_name="c", subcore_axis_name="s")
    return pl.kernel(
        _scatter_body,
        out_type=jax.ShapeDtypeStruct((NC, N_PAD, IN_C), jnp.float32),
        mesh=mesh,
        scratch_types=[
            pltpu.VMEM((CHUNK,), jnp.int32),
            pltpu.VMEM((CHUNK, IN_C), jnp.float32),
            pltpu.VMEM_SHARED((N_PAD, IN_C), jnp.float32),
            pltpu.SemaphoreType.DMA,
        ],
    )(msg2, dst, zeros_hbm)


BE = 2000


def _msg_body(xj_ref, a5_ref, w_ref, out_ref):
    acc = jnp.zeros((BE, OUT_C), jnp.float32)
    for d in range(D_EDGE + 1):
        y = jnp.dot(xj_ref[...], w_ref[d],
                    preferred_element_type=jnp.float32)
        acc = acc + a5_ref[:, d:d + 1] * y
    out_ref[:, :OUT_C] = acc
    out_ref[:, OUT_C:] = jnp.zeros((BE, IN_C - OUT_C), jnp.float32)


def _tc_msg(x_j, a5, wstk):
    return pl.pallas_call(
        _msg_body,
        grid=(E // BE,),
        in_specs=[
            pl.BlockSpec((BE, IN_C), lambda e: (e, 0)),
            pl.BlockSpec((BE, D_EDGE + 1), lambda e: (e, 0)),
            pl.BlockSpec((D_EDGE + 1, IN_C, OUT_C), lambda e: (0, 0, 0)),
        ],
        out_specs=pl.BlockSpec((BE, IN_C), lambda e: (e, 0)),
        out_shape=jax.ShapeDtypeStruct((E, IN_C), jnp.float32),
    )(x_j, a5, wstk)


BN = 1000


def _combine_body(p_ref, x_ref, w_ref, b_ref, out_ref):
    out_ref[...] = (p_ref[0, :, :OUT_C] + p_ref[1, :, :OUT_C]
                    + jnp.dot(x_ref[...], w_ref[...],
                              preferred_element_type=jnp.float32)
                    + b_ref[...])


def _tc_combine(parts, x, w_root, bias2):
    return pl.pallas_call(
        _combine_body,
        grid=(N // BN,),
        in_specs=[
            pl.BlockSpec((NC, BN, IN_C), lambda i: (0, i, 0)),
            pl.BlockSpec((BN, IN_C), lambda i: (i, 0)),
            pl.BlockSpec((IN_C, OUT_C), lambda i: (0, 0)),
            pl.BlockSpec((1, OUT_C), lambda i: (0, 0)),
        ],
        out_specs=pl.BlockSpec((BN, OUT_C), lambda i: (i, 0)),
        out_shape=jax.ShapeDtypeStruct((N, OUT_C), jnp.float32),
    )(parts, x, w_root, bias2)


@jax.jit
def _run(x, edge_index, edge_attr, M, b_edge, W_root, bias):
    src = edge_index[0]
    dst = edge_index[1]
    wstk = jnp.concatenate(
        [M.reshape(D_EDGE, IN_C, OUT_C),
         b_edge.reshape(1, IN_C, OUT_C)], axis=0)
    a5 = jnp.concatenate(
        [edge_attr, jnp.ones((E, 1), jnp.float32)], axis=1)
    zeros_hbm = jnp.zeros((N_PAD, IN_C), jnp.float32)

    x_j = _sc_gather(x, src)
    msg2 = _tc_msg(x_j, a5, wstk)
    parts = _sc_scatter(msg2, dst, zeros_hbm)
    out = _tc_combine(parts, x, W_root, bias.reshape(1, OUT_C))
    return out


def kernel(x, edge_index, edge_attr, M, b_edge, W_root, bias):
    out = _run(x, edge_index, edge_attr, M, b_edge, W_root, bias)
    return (out, edge_index, edge_attr)


# speedup vs baseline: 3.1019x; 1.1432x over previous
import jax
import jax.numpy as jnp
from jax import lax
from jax.experimental import pallas as pl
from jax.experimental.pallas import tpu as pltpu
from jax.experimental.pallas import tpu_sc as plsc

N = 10000
E = 160000
IN_C = 128
OUT_C = 64
D_EDGE = 4

NC, NS = 2, 16
NW = NC * NS
EPW = E // NW
CH = 200
TRIPS = EPW // CH
SPLIT = 128
N_PAD = 10240
STRIPE = N_PAD // NS


def _gather_body(x_hbm, src_hbm, out_hbm,
                 idx0, idx1, rows0, rows1, semi0, semi1, semg, semw0, semw1):
    wid = lax.axis_index("s") * NC + lax.axis_index("c")
    base0 = wid * EPW
    idx = (idx0, idx1)
    rows = (rows0, rows1)
    semi = (semi0, semi1)
    semw = (semw0, semw1)

    def start_idx(t, b):
        pltpu.async_copy(src_hbm.at[pl.ds(base0 + t * CH, CH)], idx[b], semi[b])

    def wait_idx(t, b):
        pltpu.make_async_copy(src_hbm.at[pl.ds(base0 + t * CH, CH)],
                              idx[b], semi[b]).wait()

    def fire_gather(b):
        c1 = pltpu.async_copy(x_hbm.at[idx[b].at[pl.ds(0, SPLIT)]],
                              rows[b].at[pl.ds(0, SPLIT)], semg)
        c2 = pltpu.async_copy(x_hbm.at[idx[b].at[pl.ds(SPLIT, CH - SPLIT)]],
                              rows[b].at[pl.ds(SPLIT, CH - SPLIT)], semg)
        c1.wait()
        c2.wait()

    def start_wb(t, b):
        pltpu.async_copy(rows[b], out_hbm.at[pl.ds(base0 + t * CH, CH)], semw[b])

    def wait_wb(t, b):
        pltpu.make_async_copy(rows[b], out_hbm.at[pl.ds(base0 + t * CH, CH)],
                              semw[b]).wait()

    start_idx(0, 0)
    wait_idx(0, 0)
    start_idx(1, 1)
    fire_gather(0)
    start_wb(0, 0)
    wait_idx(1, 1)
    start_idx(2, 0)
    fire_gather(1)
    start_wb(1, 1)

    def pair(u, _):
        for b in range(2):
            t = 2 * u + b
            wait_idx(t, b)
            wait_wb(t - 2, b)
            start_idx(t + 1, 1 - b)
            fire_gather(b)
            start_wb(t, b)
        return _

    lax.fori_loop(1, (TRIPS - 1) // 2, pair, None)

    t = TRIPS - 1
    wait_idx(t, 0)
    wait_wb(t - 2, 0)
    fire_gather(0)
    start_wb(t, 0)
    wait_wb(t - 1, 1)
    wait_wb(t, 0)


def _sc_gather(x, src):
    mesh = plsc.VectorSubcoreMesh(core_axis_name="c", subcore_axis_name="s")
    return pl.kernel(
        _gather_body,
        out_type=jax.ShapeDtypeStruct((E, IN_C), jnp.float32),
        mesh=mesh,
        scratch_types=[
            pltpu.VMEM((CH,), jnp.int32),
            pltpu.VMEM((CH,), jnp.int32),
            pltpu.VMEM((CH, IN_C), jnp.float32),
            pltpu.VMEM((CH, IN_C), jnp.float32),
            pltpu.SemaphoreType.DMA,
            pltpu.SemaphoreType.DMA,
            pltpu.SemaphoreType.DMA,
            pltpu.SemaphoreType.DMA,
            pltpu.SemaphoreType.DMA,
        ],
    )(x, src)


SCH = 128
S_NCH = E // SCH
S_TRIPS = (S_NCH + NW - 1) // NW


def _scatter_body(msg_hbm, dst_hbm, zeros_hbm, out_hbm,
                  idx0, idx1, stag0, stag1, acc_sh,
                  semi0, semi1, semr0, semr1, sems):
    c = lax.axis_index("c")
    s = lax.axis_index("s")
    wid = s * NC + c
    idx = (idx0, idx1)
    stag = (stag0, stag1)
    semi = (semi0, semi1)
    semr = (semr0, semr1)

    def cid_of(t):
        return wid + t * NW

    def start_loads(t, b):
        cid = cid_of(t)

        @pl.when(cid < S_NCH)
        def _():
            base = cid * SCH
            pltpu.async_copy(dst_hbm.at[pl.ds(base, SCH)], idx[b], semi[b])
            pltpu.async_copy(msg_hbm.at[pl.ds(base, SCH)], stag[b], semr[b])

    def trip(t, b):
        cid = cid_of(t)

        @pl.when(cid < S_NCH)
        def _():
            base = cid * SCH
            pltpu.make_async_copy(dst_hbm.at[pl.ds(base, SCH)],
                                  idx[b], semi[b]).wait()
            pltpu.make_async_copy(msg_hbm.at[pl.ds(base, SCH)],
                                  stag[b], semr[b]).wait()

        start_loads(t + 1, 1 - b)

        @pl.when(cid < S_NCH)
        def _():
            pltpu.async_copy(stag[b], acc_sh.at[idx[b]], sems,
                             add=True).wait()

    start_loads(0, 0)
    pltpu.sync_copy(zeros_hbm.at[pl.ds(s * STRIPE, STRIPE)],
                    acc_sh.at[pl.ds(s * STRIPE, STRIPE)])
    plsc.subcore_barrier()

    def pair(u, _):
        trip(2 * u, 0)
        trip(2 * u + 1, 1)
        return _

    lax.fori_loop(0, S_TRIPS // 2, pair, None)

    plsc.subcore_barrier()
    pltpu.sync_copy(acc_sh.at[pl.ds(s * STRIPE, STRIPE)],
                    out_hbm.at[c, pl.ds(s * STRIPE, STRIPE)])


def _sc_scatter(msg, dst, zeros_hbm):
    mesh = plsc.VectorSubcoreMesh(core_axis_name="c", subcore_axis_name="s")
    return pl.kernel(
        _scatter_body,
        out_type=jax.ShapeDtypeStruct((NC, N_PAD, IN_C), jnp.float32),
        mesh=mesh,
        scratch_types=[
            pltpu.VMEM((SCH,), jnp.int32),
            pltpu.VMEM((SCH,), jnp.int32),
            pltpu.VMEM((SCH, IN_C), jnp.float32),
            pltpu.VMEM((SCH, IN_C), jnp.float32),
            pltpu.VMEM_SHARED((N_PAD, IN_C), jnp.float32),
            pltpu.SemaphoreType.DMA,
            pltpu.SemaphoreType.DMA,
            pltpu.SemaphoreType.DMA,
            pltpu.SemaphoreType.DMA,
            pltpu.SemaphoreType.DMA,
        ],
    )(msg, dst, zeros_hbm)


BE = 2000


def _msg_body(xj_ref, a5_ref, w_ref, out_ref):
    acc = jnp.zeros((BE, OUT_C), jnp.float32)
    for d in range(D_EDGE + 1):
        y = jnp.dot(xj_ref[...], w_ref[d],
                    preferred_element_type=jnp.float32)
        acc = acc + a5_ref[:, d:d + 1] * y
    out_ref[:, :OUT_C] = acc
    out_ref[:, OUT_C:] = jnp.zeros((BE, IN_C - OUT_C), jnp.float32)


def _tc_msg(x_j, a5, wstk):
    return pl.pallas_call(
        _msg_body,
        grid=(E // BE,),
        in_specs=[
            pl.BlockSpec((BE, IN_C), lambda e: (e, 0)),
            pl.BlockSpec((BE, D_EDGE + 1), lambda e: (e, 0)),
            pl.BlockSpec((D_EDGE + 1, IN_C, OUT_C), lambda e: (0, 0, 0)),
        ],
        out_specs=pl.BlockSpec((BE, IN_C), lambda e: (e, 0)),
        out_shape=jax.ShapeDtypeStruct((E, IN_C), jnp.float32),
    )(x_j, a5, wstk)


BN = 1000


def _combine_body(p_ref, x_ref, w_ref, b_ref, out_ref):
    out_ref[...] = (p_ref[0, :, :OUT_C] + p_ref[1, :, :OUT_C]
                    + jnp.dot(x_ref[...], w_ref[...],
                              preferred_element_type=jnp.float32)
                    + b_ref[...])


def _tc_combine(parts, x, w_root, bias2):
    return pl.pallas_call(
        _combine_body,
        grid=(N // BN,),
        in_specs=[
            pl.BlockSpec((NC, BN, IN_C), lambda i: (0, i, 0)),
            pl.BlockSpec((BN, IN_C), lambda i: (i, 0)),
            pl.BlockSpec((IN_C, OUT_C), lambda i: (0, 0)),
            pl.BlockSpec((1, OUT_C), lambda i: (0, 0)),
        ],
        out_specs=pl.BlockSpec((BN, OUT_C), lambda i: (i, 0)),
        out_shape=jax.ShapeDtypeStruct((N, OUT_C), jnp.float32),
    )(parts, x, w_root, bias2)


@jax.jit
def _run(x, edge_index, edge_attr, M, b_edge, W_root, bias):
    src = edge_index[0]
    dst = edge_index[1]
    wstk = jnp.concatenate(
        [M.reshape(D_EDGE, IN_C, OUT_C),
         b_edge.reshape(1, IN_C, OUT_C)], axis=0)
    a5 = jnp.concatenate(
        [edge_attr, jnp.ones((E, 1), jnp.float32)], axis=1)
    zeros_hbm = jnp.zeros((N_PAD, IN_C), jnp.float32)

    x_j = _sc_gather(x, src)
    msg = _tc_msg(x_j, a5, wstk)
    parts = _sc_scatter(msg, dst, zeros_hbm)
    out = _tc_combine(parts, x, W_root, bias.reshape(1, OUT_C))
    return out


def kernel(x, edge_index, edge_attr, M, b_edge, W_root, bias):
    out = _run(x, edge_index, edge_attr, M, b_edge, W_root, bias)
    return (out, edge_index, edge_attr)


# speedup vs baseline: 4.0304x; 1.2993x over previous
import jax
import jax.numpy as jnp
from jax import lax
from jax.experimental import pallas as pl
from jax.experimental.pallas import tpu as pltpu
from jax.experimental.pallas import tpu_sc as plsc

N = 10000
E = 160000
IN_C = 128
OUT_C = 64
D_EDGE = 4

NC, NS = 2, 16
NW = NC * NS
EPW = E // NW
CH = 200
TRIPS = EPW // CH
SPLIT = 128
N_PAD = 10240
STRIPE = N_PAD // NS


def _gather_body(x_hbm, src_hbm, out_hbm,
                 idx0, idx1, rows0, rows1, semi0, semi1, semg, semw0, semw1):
    wid = lax.axis_index("s") * NC + lax.axis_index("c")
    base0 = wid * EPW
    idx = (idx0, idx1)
    rows = (rows0, rows1)
    semi = (semi0, semi1)
    semw = (semw0, semw1)

    def start_idx(t, b):
        pltpu.async_copy(src_hbm.at[pl.ds(base0 + t * CH, CH)], idx[b], semi[b])

    def wait_idx(t, b):
        pltpu.make_async_copy(src_hbm.at[pl.ds(base0 + t * CH, CH)],
                              idx[b], semi[b]).wait()

    def fire_gather(b):
        c1 = pltpu.async_copy(x_hbm.at[idx[b].at[pl.ds(0, SPLIT)]],
                              rows[b].at[pl.ds(0, SPLIT)], semg)
        c2 = pltpu.async_copy(x_hbm.at[idx[b].at[pl.ds(SPLIT, CH - SPLIT)]],
                              rows[b].at[pl.ds(SPLIT, CH - SPLIT)], semg)
        c1.wait()
        c2.wait()

    def start_wb(t, b):
        pltpu.async_copy(rows[b], out_hbm.at[pl.ds(base0 + t * CH, CH)], semw[b])

    def wait_wb(t, b):
        pltpu.make_async_copy(rows[b], out_hbm.at[pl.ds(base0 + t * CH, CH)],
                              semw[b]).wait()

    start_idx(0, 0)
    wait_idx(0, 0)
    start_idx(1, 1)
    fire_gather(0)
    start_wb(0, 0)
    wait_idx(1, 1)
    start_idx(2, 0)
    fire_gather(1)
    start_wb(1, 1)

    def pair(u, _):
        for b in range(2):
            t = 2 * u + b
            wait_idx(t, b)
            wait_wb(t - 2, b)
            start_idx(t + 1, 1 - b)
            fire_gather(b)
            start_wb(t, b)
        return _

    lax.fori_loop(1, (TRIPS - 1) // 2, pair, None)

    t = TRIPS - 1
    wait_idx(t, 0)
    wait_wb(t - 2, 0)
    fire_gather(0)
    start_wb(t, 0)
    wait_wb(t - 1, 1)
    wait_wb(t, 0)


def _sc_gather(x, src):
    mesh = plsc.VectorSubcoreMesh(core_axis_name="c", subcore_axis_name="s")
    return pl.kernel(
        _gather_body,
        out_type=jax.ShapeDtypeStruct((E, IN_C), jnp.float32),
        mesh=mesh,
        compiler_params=pltpu.CompilerParams(use_tc_tiling_on_sc=True),
        scratch_types=[
            pltpu.VMEM((CH,), jnp.int32),
            pltpu.VMEM((CH,), jnp.int32),
            pltpu.VMEM((CH, IN_C), jnp.float32),
            pltpu.VMEM((CH, IN_C), jnp.float32),
            pltpu.SemaphoreType.DMA,
            pltpu.SemaphoreType.DMA,
            pltpu.SemaphoreType.DMA,
            pltpu.SemaphoreType.DMA,
            pltpu.SemaphoreType.DMA,
        ],
    )(x, src)


SCH = 128
S_NCH = E // SCH
S_TRIPS = (S_NCH + NW - 1) // NW


def _scatter_body(msg_hbm, dst_hbm, zeros_hbm, out_hbm,
                  idx0, idx1, stag0, stag1, acc_sh,
                  semi0, semi1, semr0, semr1, sems):
    c = lax.axis_index("c")
    s = lax.axis_index("s")
    wid = s * NC + c
    idx = (idx0, idx1)
    stag = (stag0, stag1)
    semi = (semi0, semi1)
    semr = (semr0, semr1)

    def cid_of(t):
        return wid + t * NW

    def start_loads(t, b):
        cid = cid_of(t)

        @pl.when(cid < S_NCH)
        def _():
            base = cid * SCH
            pltpu.async_copy(dst_hbm.at[pl.ds(base, SCH)], idx[b], semi[b])
            pltpu.async_copy(msg_hbm.at[pl.ds(base, SCH)], stag[b], semr[b])

    def trip(t, b):
        cid = cid_of(t)

        @pl.when(cid < S_NCH)
        def _():
            base = cid * SCH
            pltpu.make_async_copy(dst_hbm.at[pl.ds(base, SCH)],
                                  idx[b], semi[b]).wait()
            pltpu.make_async_copy(msg_hbm.at[pl.ds(base, SCH)],
                                  stag[b], semr[b]).wait()

        start_loads(t + 1, 1 - b)

        @pl.when(cid < S_NCH)
        def _():
            pltpu.async_copy(stag[b], acc_sh.at[idx[b]], sems,
                             add=True).wait()

    start_loads(0, 0)
    pltpu.sync_copy(zeros_hbm.at[pl.ds(s * STRIPE, STRIPE)],
                    acc_sh.at[pl.ds(s * STRIPE, STRIPE)])
    plsc.subcore_barrier()

    def pair(u, _):
        trip(2 * u, 0)
        trip(2 * u + 1, 1)
        return _

    lax.fori_loop(0, S_TRIPS // 2, pair, None)

    plsc.subcore_barrier()
    pltpu.sync_copy(acc_sh.at[pl.ds(s * STRIPE, STRIPE)],
                    out_hbm.at[c, pl.ds(s * STRIPE, STRIPE)])


def _sc_scatter(msg, dst, zeros_hbm):
    mesh = plsc.VectorSubcoreMesh(core_axis_name="c", subcore_axis_name="s")
    return pl.kernel(
        _scatter_body,
        out_type=jax.ShapeDtypeStruct((NC, N_PAD, IN_C), jnp.float32),
        mesh=mesh,
        compiler_params=pltpu.CompilerParams(use_tc_tiling_on_sc=True),
        scratch_types=[
            pltpu.VMEM((SCH,), jnp.int32),
            pltpu.VMEM((SCH,), jnp.int32),
            pltpu.VMEM((SCH, IN_C), jnp.float32),
            pltpu.VMEM((SCH, IN_C), jnp.float32),
            pltpu.VMEM_SHARED((N_PAD, IN_C), jnp.float32),
            pltpu.SemaphoreType.DMA,
            pltpu.SemaphoreType.DMA,
            pltpu.SemaphoreType.DMA,
            pltpu.SemaphoreType.DMA,
            pltpu.SemaphoreType.DMA,
        ],
    )(msg, dst, zeros_hbm)


BE = 3200


def _msg_body(xj_ref, at_ref, w_ref, out_ref):
    xj = xj_ref[...]
    a = at_ref[...].T
    acc = jnp.dot(xj, w_ref[D_EDGE],
                  preferred_element_type=jnp.float32)
    for d in range(D_EDGE):
        y = jnp.dot(xj, w_ref[d], preferred_element_type=jnp.float32)
        acc = acc + a[:, d:d + 1] * y
    out_ref[:, :OUT_C] = acc
    out_ref[:, OUT_C:] = jnp.zeros((BE, IN_C - OUT_C), jnp.float32)


def _tc_msg(x_j, a_t, wstk):
    return pl.pallas_call(
        _msg_body,
        grid=(E // BE,),
        in_specs=[
            pl.BlockSpec((BE, IN_C), lambda e: (e, 0)),
            pl.BlockSpec((D_EDGE, BE), lambda e: (0, e)),
            pl.BlockSpec((D_EDGE + 1, IN_C, OUT_C), lambda e: (0, 0, 0)),
        ],
        out_specs=pl.BlockSpec((BE, IN_C), lambda e: (e, 0)),
        out_shape=jax.ShapeDtypeStruct((E, IN_C), jnp.float32),
    )(x_j, a_t, wstk)


BN = 1000


def _combine_body(p_ref, x_ref, w_ref, b_ref, out_ref):
    out_ref[...] = (p_ref[0, :, :OUT_C] + p_ref[1, :, :OUT_C]
                    + jnp.dot(x_ref[...], w_ref[...],
                              preferred_element_type=jnp.float32)
                    + b_ref[...])


def _tc_combine(parts, x, w_root, bias2):
    return pl.pallas_call(
        _combine_body,
        grid=(N // BN,),
        in_specs=[
            pl.BlockSpec((NC, BN, IN_C), lambda i: (0, i, 0)),
            pl.BlockSpec((BN, IN_C), lambda i: (i, 0)),
            pl.BlockSpec((IN_C, OUT_C), lambda i: (0, 0)),
            pl.BlockSpec((1, OUT_C), lambda i: (0, 0)),
        ],
        out_specs=pl.BlockSpec((BN, OUT_C), lambda i: (i, 0)),
        out_shape=jax.ShapeDtypeStruct((N, OUT_C), jnp.float32),
    )(parts, x, w_root, bias2)


@jax.jit
def _run(x, edge_index, edge_attr, M, b_edge, W_root, bias):
    src = edge_index[0]
    dst = edge_index[1]
    wstk = jnp.concatenate(
        [M.reshape(D_EDGE, IN_C, OUT_C),
         b_edge.reshape(1, IN_C, OUT_C)], axis=0)
    a_t = edge_attr.T
    zeros_hbm = jnp.zeros((N_PAD, IN_C), jnp.float32)

    x_j = _sc_gather(x, src)
    msg = _tc_msg(x_j, a_t, wstk)
    parts = _sc_scatter(msg, dst, zeros_hbm)
    out = _tc_combine(parts, x, W_root, bias.reshape(1, OUT_C))
    return out


def kernel(x, edge_index, edge_attr, M, b_edge, W_root, bias):
    out = _run(x, edge_index, edge_attr, M, b_edge, W_root, bias)
    return (out, edge_index, edge_attr)


# speedup vs baseline: 4.2853x; 1.0633x over previous
import jax
import jax.numpy as jnp
from jax import lax
from jax.experimental import pallas as pl
from jax.experimental.pallas import tpu as pltpu
from jax.experimental.pallas import tpu_sc as plsc

N = 10000
E = 160000
IN_C = 128
OUT_C = 64
D_EDGE = 4

NC, NS = 2, 16
NW = NC * NS
EPW = E // NW
CH = 200
TRIPS = EPW // CH
SPLIT = 128
N_PAD = 10240
STRIPE = N_PAD // NS


def _gather_body(x_hbm, src_hbm, out_hbm,
                 idx0, idx1, rows0, rows1, semi0, semi1, semg, semw0, semw1):
    wid = lax.axis_index("s") * NC + lax.axis_index("c")
    base0 = wid * EPW
    idx = (idx0, idx1)
    rows = (rows0, rows1)
    semi = (semi0, semi1)
    semw = (semw0, semw1)

    def start_idx(t, b):
        pltpu.async_copy(src_hbm.at[pl.ds(base0 + t * CH, CH)], idx[b], semi[b])

    def wait_idx(t, b):
        pltpu.make_async_copy(src_hbm.at[pl.ds(base0 + t * CH, CH)],
                              idx[b], semi[b]).wait()

    def fire_gather(b):
        c1 = pltpu.async_copy(x_hbm.at[idx[b].at[pl.ds(0, SPLIT)]],
                              rows[b].at[pl.ds(0, SPLIT)], semg)
        c2 = pltpu.async_copy(x_hbm.at[idx[b].at[pl.ds(SPLIT, CH - SPLIT)]],
                              rows[b].at[pl.ds(SPLIT, CH - SPLIT)], semg)
        c1.wait()
        c2.wait()

    def start_wb(t, b):
        pltpu.async_copy(rows[b], out_hbm.at[pl.ds(base0 + t * CH, CH)], semw[b])

    def wait_wb(t, b):
        pltpu.make_async_copy(rows[b], out_hbm.at[pl.ds(base0 + t * CH, CH)],
                              semw[b]).wait()

    start_idx(0, 0)
    wait_idx(0, 0)
    start_idx(1, 1)
    fire_gather(0)
    start_wb(0, 0)
    wait_idx(1, 1)
    start_idx(2, 0)
    fire_gather(1)
    start_wb(1, 1)

    def pair(u, _):
        for b in range(2):
            t = 2 * u + b
            wait_idx(t, b)
            wait_wb(t - 2, b)
            start_idx(t + 1, 1 - b)
            fire_gather(b)
            start_wb(t, b)
        return _

    lax.fori_loop(1, (TRIPS - 1) // 2, pair, None)

    t = TRIPS - 1
    wait_idx(t, 0)
    wait_wb(t - 2, 0)
    fire_gather(0)
    start_wb(t, 0)
    wait_wb(t - 1, 1)
    wait_wb(t, 0)


def _sc_gather(x, src):
    mesh = plsc.VectorSubcoreMesh(core_axis_name="c", subcore_axis_name="s")
    return pl.kernel(
        _gather_body,
        out_type=jax.ShapeDtypeStruct((E, IN_C), jnp.float32),
        mesh=mesh,
        compiler_params=pltpu.CompilerParams(use_tc_tiling_on_sc=True),
        scratch_types=[
            pltpu.VMEM((CH,), jnp.int32),
            pltpu.VMEM((CH,), jnp.int32),
            pltpu.VMEM((CH, IN_C), jnp.float32),
            pltpu.VMEM((CH, IN_C), jnp.float32),
            pltpu.SemaphoreType.DMA,
            pltpu.SemaphoreType.DMA,
            pltpu.SemaphoreType.DMA,
            pltpu.SemaphoreType.DMA,
            pltpu.SemaphoreType.DMA,
        ],
    )(x, src)


SCH = 128
S_NCH = E // SCH
S_TRIPS = (S_NCH + NW - 1) // NW


def _scatter_body(msg_hbm, dst_hbm, zeros_hbm, out_hbm,
                  idx0, idx1, stag0, stag1, acc_sh,
                  semi0, semi1, semr0, semr1, sems):
    c = lax.axis_index("c")
    s = lax.axis_index("s")
    wid = s * NC + c
    idx = (idx0, idx1)
    stag = (stag0, stag1)
    semi = (semi0, semi1)
    semr = (semr0, semr1)

    def cid_of(t):
        return wid + t * NW

    def start_loads(t, b):
        cid = cid_of(t)

        @pl.when(cid < S_NCH)
        def _():
            base = cid * SCH
            pltpu.async_copy(dst_hbm.at[pl.ds(base, SCH)], idx[b], semi[b])
            pltpu.async_copy(msg_hbm.at[pl.ds(base, SCH)], stag[b], semr[b])

    def trip(t, b):
        cid = cid_of(t)

        @pl.when(cid < S_NCH)
        def _():
            base = cid * SCH
            pltpu.make_async_copy(dst_hbm.at[pl.ds(base, SCH)],
                                  idx[b], semi[b]).wait()
            pltpu.make_async_copy(msg_hbm.at[pl.ds(base, SCH)],
                                  stag[b], semr[b]).wait()

        start_loads(t + 1, 1 - b)

        @pl.when(cid < S_NCH)
        def _():
            pltpu.async_copy(stag[b], acc_sh.at[idx[b]], sems,
                             add=True).wait()

    start_loads(0, 0)
    pltpu.sync_copy(zeros_hbm.at[pl.ds(s * STRIPE, STRIPE)],
                    acc_sh.at[pl.ds(s * STRIPE, STRIPE)])
    plsc.subcore_barrier()

    def pair(u, _):
        trip(2 * u, 0)
        trip(2 * u + 1, 1)
        return _

    lax.fori_loop(0, S_TRIPS // 2, pair, None)

    plsc.subcore_barrier()
    pltpu.sync_copy(acc_sh.at[pl.ds(s * STRIPE, STRIPE)],
                    out_hbm.at[c, pl.ds(s * STRIPE, STRIPE)])


def _sc_scatter(msg, dst, zeros_hbm):
    mesh = plsc.VectorSubcoreMesh(core_axis_name="c", subcore_axis_name="s")
    return pl.kernel(
        _scatter_body,
        out_type=jax.ShapeDtypeStruct((NC, N_PAD, IN_C), jnp.float32),
        mesh=mesh,
        compiler_params=pltpu.CompilerParams(use_tc_tiling_on_sc=True),
        scratch_types=[
            pltpu.VMEM((SCH,), jnp.int32),
            pltpu.VMEM((SCH,), jnp.int32),
            pltpu.VMEM((SCH, IN_C), jnp.float32),
            pltpu.VMEM((SCH, IN_C), jnp.float32),
            pltpu.VMEM_SHARED((N_PAD, IN_C), jnp.float32),
            pltpu.SemaphoreType.DMA,
            pltpu.SemaphoreType.DMA,
            pltpu.SemaphoreType.DMA,
            pltpu.SemaphoreType.DMA,
            pltpu.SemaphoreType.DMA,
        ],
    )(msg, dst, zeros_hbm)


BE = 3200


def _msg_body(xj_ref, at_ref, w_ref, out_ref):
    xj = xj_ref[...]
    dn = (((0,), (1,)), ((), ()))
    acc = lax.dot_general(w_ref[D_EDGE], xj, dn,
                          preferred_element_type=jnp.float32)
    for d in range(D_EDGE):
        y = lax.dot_general(w_ref[d], xj, dn,
                            preferred_element_type=jnp.float32)
        ad = jnp.broadcast_to(at_ref[d][None, :], (OUT_C, BE))
        acc = acc + ad * y
    out_ref[:, :OUT_C] = acc.T


def _tc_msg(x_j, a_t, wstk):
    return pl.pallas_call(
        _msg_body,
        grid=(E // BE,),
        in_specs=[
            pl.BlockSpec((BE, IN_C), lambda e: (e, 0)),
            pl.BlockSpec((D_EDGE, BE), lambda e: (0, e)),
            pl.BlockSpec((D_EDGE + 1, IN_C, OUT_C), lambda e: (0, 0, 0)),
        ],
        out_specs=pl.BlockSpec((BE, IN_C), lambda e: (e, 0)),
        out_shape=jax.ShapeDtypeStruct((E, IN_C), jnp.float32),
    )(x_j, a_t, wstk)


BN = 1000


def _combine_body(p_ref, x_ref, w_ref, b_ref, out_ref):
    out_ref[...] = (p_ref[0, :, :OUT_C] + p_ref[1, :, :OUT_C]
                    + jnp.dot(x_ref[...], w_ref[...],
                              preferred_element_type=jnp.float32)
                    + b_ref[...])


def _tc_combine(parts, x, w_root, bias2):
    return pl.pallas_call(
        _combine_body,
        grid=(N // BN,),
        in_specs=[
            pl.BlockSpec((NC, BN, IN_C), lambda i: (0, i, 0)),
            pl.BlockSpec((BN, IN_C), lambda i: (i, 0)),
            pl.BlockSpec((IN_C, OUT_C), lambda i: (0, 0)),
            pl.BlockSpec((1, OUT_C), lambda i: (0, 0)),
        ],
        out_specs=pl.BlockSpec((BN, OUT_C), lambda i: (i, 0)),
        out_shape=jax.ShapeDtypeStruct((N, OUT_C), jnp.float32),
    )(parts, x, w_root, bias2)


@jax.jit
def _run(x, edge_index, edge_attr, M, b_edge, W_root, bias):
    src = edge_index[0]
    dst = edge_index[1]
    wstk = jnp.concatenate(
        [M.reshape(D_EDGE, IN_C, OUT_C),
         b_edge.reshape(1, IN_C, OUT_C)], axis=0)
    a_t = edge_attr.T
    zeros_hbm = jnp.zeros((N_PAD, IN_C), jnp.float32)

    x_j = _sc_gather(x, src)
    msg = _tc_msg(x_j, a_t, wstk)
    parts = _sc_scatter(msg, dst, zeros_hbm)
    out = _tc_combine(parts, x, W_root, bias.reshape(1, OUT_C))
    return out


def kernel(x, edge_index, edge_attr, M, b_edge, W_root, bias):
    out = _run(x, edge_index, edge_attr, M, b_edge, W_root, bias)
    return (out, edge_index, edge_attr)


# speedup vs baseline: 4.8506x; 1.1319x over previous
import jax
import jax.numpy as jnp
from jax import lax
from jax.experimental import pallas as pl
from jax.experimental.pallas import tpu as pltpu
from jax.experimental.pallas import tpu_sc as plsc

N = 10000
E = 160000
IN_C = 128
OUT_C = 64
D_EDGE = 4

NC, NS = 2, 16
NW = NC * NS
EPW = E // NW
CH = 200
TRIPS = EPW // CH
SPLIT = 128
N_PAD = 10240
STRIPE = N_PAD // NS


EH = E // 2
SCH_G = 128
G_NCH = EH // SCH_G
G_TRIPS = (G_NCH + NW - 1) // NW


def _gather_body(x_hbm, src_hbm, out_hbm,
                 idx0, idx1, rows0, rows1, semi0, semi1, semg, semw0, semw1):
    wid = lax.axis_index("s") * NC + lax.axis_index("c")
    idx = (idx0, idx1)
    rows = (rows0, rows1)
    semi = (semi0, semi1)
    semw = (semw0, semw1)

    def cid_of(t):
        return wid + t * NW

    def start_idx(t, b):
        cid = cid_of(t)

        @pl.when(cid < G_NCH)
        def _():
            pltpu.async_copy(src_hbm.at[pl.ds(cid * SCH_G, SCH_G)],
                             idx[b], semi[b])

    def trip(t, b):
        cid = cid_of(t)

        @pl.when(cid < G_NCH)
        def _():
            pltpu.make_async_copy(src_hbm.at[pl.ds(cid * SCH_G, SCH_G)],
                                  idx[b], semi[b]).wait()

        @pl.when((t >= 2) & (cid_of(t - 2) < G_NCH))
        def _():
            pltpu.make_async_copy(
                rows[b], out_hbm.at[pl.ds(cid_of(t - 2) * SCH_G, SCH_G)],
                semw[b]).wait()

        start_idx(t + 1, 1 - b)

        @pl.when(cid < G_NCH)
        def _():
            pltpu.async_copy(x_hbm.at[idx[b]], rows[b], semg).wait()
            pltpu.async_copy(rows[b], out_hbm.at[pl.ds(cid * SCH_G, SCH_G)],
                             semw[b])

    start_idx(0, 0)

    def pairs(u, _):
        trip(2 * u, 0)
        trip(2 * u + 1, 1)
        return _

    lax.fori_loop(0, G_TRIPS // 2, pairs, None)

    for tl in (G_TRIPS - 2, G_TRIPS - 1):
        cid = cid_of(tl)

        @pl.when(cid < G_NCH)
        def _():
            pltpu.make_async_copy(
                rows[tl % 2], out_hbm.at[pl.ds(cid * SCH_G, SCH_G)],
                semw[tl % 2]).wait()


def _sc_gather(x, src_half):
    mesh = plsc.VectorSubcoreMesh(core_axis_name="c", subcore_axis_name="s")
    return pl.kernel(
        _gather_body,
        out_type=jax.ShapeDtypeStruct((EH, IN_C), jnp.float32),
        mesh=mesh,
        compiler_params=pltpu.CompilerParams(use_tc_tiling_on_sc=True),
        scratch_types=[
            pltpu.VMEM((SCH_G,), jnp.int32),
            pltpu.VMEM((SCH_G,), jnp.int32),
            pltpu.VMEM((SCH_G, IN_C), jnp.float32),
            pltpu.VMEM((SCH_G, IN_C), jnp.float32),
            pltpu.SemaphoreType.DMA,
            pltpu.SemaphoreType.DMA,
            pltpu.SemaphoreType.DMA,
            pltpu.SemaphoreType.DMA,
            pltpu.SemaphoreType.DMA,
        ],
    )(x, src_half)


SCH = 128
S_NCH = EH // SCH
S_TRIPS = (S_NCH + NW - 1) // NW


def _scatter_body(msg_hbm, dst_hbm, zeros_hbm, out_hbm,
                  idx0, idx1, stag0, stag1, acc_sh,
                  semi0, semi1, semr0, semr1, sems):
    c = lax.axis_index("c")
    s = lax.axis_index("s")
    wid = s * NC + c
    idx = (idx0, idx1)
    stag = (stag0, stag1)
    semi = (semi0, semi1)
    semr = (semr0, semr1)

    def cid_of(t):
        return wid + t * NW

    def start_loads(t, b):
        cid = cid_of(t)

        @pl.when(cid < S_NCH)
        def _():
            base = cid * SCH
            pltpu.async_copy(dst_hbm.at[pl.ds(base, SCH)], idx[b], semi[b])
            pltpu.async_copy(msg_hbm.at[pl.ds(base, SCH)], stag[b], semr[b])

    def trip(t, b):
        cid = cid_of(t)

        @pl.when(cid < S_NCH)
        def _():
            base = cid * SCH
            pltpu.make_async_copy(dst_hbm.at[pl.ds(base, SCH)],
                                  idx[b], semi[b]).wait()
            pltpu.make_async_copy(msg_hbm.at[pl.ds(base, SCH)],
                                  stag[b], semr[b]).wait()

        start_loads(t + 1, 1 - b)

        @pl.when(cid < S_NCH)
        def _():
            pltpu.async_copy(stag[b], acc_sh.at[idx[b]], sems,
                             add=True).wait()

    start_loads(0, 0)
    pltpu.sync_copy(zeros_hbm.at[pl.ds(s * STRIPE, STRIPE)],
                    acc_sh.at[pl.ds(s * STRIPE, STRIPE)])
    plsc.subcore_barrier()

    def pair(u, _):
        trip(2 * u, 0)
        trip(2 * u + 1, 1)
        return _

    lax.fori_loop(0, S_TRIPS // 2, pair, None)

    plsc.subcore_barrier()
    pltpu.sync_copy(acc_sh.at[pl.ds(s * STRIPE, STRIPE)],
                    out_hbm.at[c, pl.ds(s * STRIPE, STRIPE)])


def _sc_scatter(msg, dst, zeros_hbm):
    mesh = plsc.VectorSubcoreMesh(core_axis_name="c", subcore_axis_name="s")
    return pl.kernel(
        _scatter_body,
        out_type=jax.ShapeDtypeStruct((NC, N_PAD, IN_C), jnp.float32),
        mesh=mesh,
        compiler_params=pltpu.CompilerParams(use_tc_tiling_on_sc=True),
        scratch_types=[
            pltpu.VMEM((SCH,), jnp.int32),
            pltpu.VMEM((SCH,), jnp.int32),
            pltpu.VMEM((SCH, IN_C), jnp.float32),
            pltpu.VMEM((SCH, IN_C), jnp.float32),
            pltpu.VMEM_SHARED((N_PAD, IN_C), jnp.float32),
            pltpu.SemaphoreType.DMA,
            pltpu.SemaphoreType.DMA,
            pltpu.SemaphoreType.DMA,
            pltpu.SemaphoreType.DMA,
            pltpu.SemaphoreType.DMA,
        ],
    )(msg, dst, zeros_hbm)


BE = 3200


def _msg_body(xj_ref, at_ref, w_ref, out_ref):
    xj = xj_ref[...]
    dn = (((0,), (1,)), ((), ()))
    acc = lax.dot_general(w_ref[D_EDGE], xj, dn,
                          preferred_element_type=jnp.float32)
    for d in range(D_EDGE):
        y = lax.dot_general(w_ref[d], xj, dn,
                            preferred_element_type=jnp.float32)
        ad = jnp.broadcast_to(at_ref[d][None, :], (OUT_C, BE))
        acc = acc + ad * y
    out_ref[:, :OUT_C] = acc.T


def _tc_msg(x_j, a_t, wstk):
    return pl.pallas_call(
        _msg_body,
        grid=(EH // BE,),
        in_specs=[
            pl.BlockSpec((BE, IN_C), lambda e: (e, 0)),
            pl.BlockSpec((D_EDGE, BE), lambda e: (0, e)),
            pl.BlockSpec((D_EDGE + 1, IN_C, OUT_C), lambda e: (0, 0, 0)),
        ],
        out_specs=pl.BlockSpec((BE, IN_C), lambda e: (e, 0)),
        out_shape=jax.ShapeDtypeStruct((EH, IN_C), jnp.float32),
    )(x_j, a_t, wstk)


BN = 1000


def _combine_body(p0_ref, p1_ref, x_ref, w_ref, b_ref, out_ref):
    out_ref[...] = (p0_ref[0, :, :OUT_C] + p0_ref[1, :, :OUT_C]
                    + p1_ref[0, :, :OUT_C] + p1_ref[1, :, :OUT_C]
                    + jnp.dot(x_ref[...], w_ref[...],
                              preferred_element_type=jnp.float32)
                    + b_ref[...])


def _tc_combine(parts0, parts1, x, w_root, bias2):
    return pl.pallas_call(
        _combine_body,
        grid=(N // BN,),
        in_specs=[
            pl.BlockSpec((NC, BN, IN_C), lambda i: (0, i, 0)),
            pl.BlockSpec((NC, BN, IN_C), lambda i: (0, i, 0)),
            pl.BlockSpec((BN, IN_C), lambda i: (i, 0)),
            pl.BlockSpec((IN_C, OUT_C), lambda i: (0, 0)),
            pl.BlockSpec((1, OUT_C), lambda i: (0, 0)),
        ],
        out_specs=pl.BlockSpec((BN, OUT_C), lambda i: (i, 0)),
        out_shape=jax.ShapeDtypeStruct((N, OUT_C), jnp.float32),
    )(parts0, parts1, x, w_root, bias2)


@jax.jit
def _run(x, edge_index, edge_attr, M, b_edge, W_root, bias):
    src = edge_index[0]
    dst = edge_index[1]
    wstk = jnp.concatenate(
        [M.reshape(D_EDGE, IN_C, OUT_C),
         b_edge.reshape(1, IN_C, OUT_C)], axis=0)
    a_t = edge_attr.T
    zeros_hbm = jnp.zeros((N_PAD, IN_C), jnp.float32)

    xj0 = _sc_gather(x, src[:EH])
    xj1 = _sc_gather(x, src[EH:])
    msg0 = _tc_msg(xj0, a_t[:, :EH], wstk)
    msg1 = _tc_msg(xj1, a_t[:, EH:], wstk)
    parts0 = _sc_scatter(msg0, dst[:EH], zeros_hbm)
    parts1 = _sc_scatter(msg1, dst[EH:], zeros_hbm)
    out = _tc_combine(parts0, parts1, x, W_root, bias.reshape(1, OUT_C))
    return out


def kernel(x, edge_index, edge_attr, M, b_edge, W_root, bias):
    out = _run(x, edge_index, edge_attr, M, b_edge, W_root, bias)
    return (out, edge_index, edge_attr)


# speedup vs baseline: 4.9457x; 1.0196x over previous
import jax
import jax.numpy as jnp
from jax import lax
from jax.experimental import pallas as pl
from jax.experimental.pallas import tpu as pltpu
from jax.experimental.pallas import tpu_sc as plsc

N = 10000
E = 160000
IN_C = 128
OUT_C = 64
D_EDGE = 4

NC, NS = 2, 16
NW = NC * NS
EPW = E // NW
CH = 200
TRIPS = EPW // CH
SPLIT = 128
N_PAD = 10240
STRIPE = N_PAD // NS


EH = E // 2
SCH_G = 128
G_NCH = EH // SCH_G
G_TRIPS = (G_NCH + NW - 1) // NW


def _gather_body(x_hbm, src_hbm, out_hbm,
                 idx0, idx1, rows0, rows1, semi0, semi1, semg, semw0, semw1):
    wid = lax.axis_index("s") * NC + lax.axis_index("c")
    idx = (idx0, idx1)
    rows = (rows0, rows1)
    semi = (semi0, semi1)
    semw = (semw0, semw1)

    def cid_of(t):
        return wid + t * NW

    def start_idx(t, b):
        cid = cid_of(t)

        @pl.when(cid < G_NCH)
        def _():
            pltpu.async_copy(src_hbm.at[pl.ds(cid * SCH_G, SCH_G)],
                             idx[b], semi[b])

    def trip(t, b):
        cid = cid_of(t)

        @pl.when(cid < G_NCH)
        def _():
            pltpu.make_async_copy(src_hbm.at[pl.ds(cid * SCH_G, SCH_G)],
                                  idx[b], semi[b]).wait()

        @pl.when((t >= 2) & (cid_of(t - 2) < G_NCH))
        def _():
            pltpu.make_async_copy(
                rows[b], out_hbm.at[pl.ds(cid_of(t - 2) * SCH_G, SCH_G)],
                semw[b]).wait()

        start_idx(t + 1, 1 - b)

        @pl.when(cid < G_NCH)
        def _():
            pltpu.async_copy(x_hbm.at[idx[b]], rows[b], semg).wait()
            pltpu.async_copy(rows[b], out_hbm.at[pl.ds(cid * SCH_G, SCH_G)],
                             semw[b])

    start_idx(0, 0)

    def pairs(u, _):
        trip(2 * u, 0)
        trip(2 * u + 1, 1)
        return _

    lax.fori_loop(0, G_TRIPS // 2, pairs, None)

    for tl in (G_TRIPS - 2, G_TRIPS - 1):
        cid = cid_of(tl)

        @pl.when(cid < G_NCH)
        def _():
            pltpu.make_async_copy(
                rows[tl % 2], out_hbm.at[pl.ds(cid * SCH_G, SCH_G)],
                semw[tl % 2]).wait()


def _sc_gather(x, src_half):
    mesh = plsc.VectorSubcoreMesh(core_axis_name="c", subcore_axis_name="s")
    return pl.kernel(
        _gather_body,
        out_type=jax.ShapeDtypeStruct((EH, IN_C), jnp.float32),
        mesh=mesh,
        compiler_params=pltpu.CompilerParams(use_tc_tiling_on_sc=True),
        scratch_types=[
            pltpu.VMEM((SCH_G,), jnp.int32),
            pltpu.VMEM((SCH_G,), jnp.int32),
            pltpu.VMEM((SCH_G, IN_C), jnp.float32),
            pltpu.VMEM((SCH_G, IN_C), jnp.float32),
            pltpu.SemaphoreType.DMA,
            pltpu.SemaphoreType.DMA,
            pltpu.SemaphoreType.DMA,
            pltpu.SemaphoreType.DMA,
            pltpu.SemaphoreType.DMA,
        ],
    )(x, src_half)


SCH = 128
S_NCH = EH // SCH
S_TRIPS = (S_NCH + NW - 1) // NW


def _scatter_body(msg_hbm, dst_hbm, zeros_hbm, out_hbm,
                  idx0, idx1, stag0, stag1, acc_sh,
                  semi0, semi1, semr0, semr1, sems):
    c = lax.axis_index("c")
    s = lax.axis_index("s")
    wid = s * NC + c
    idx = (idx0, idx1)
    stag = (stag0, stag1)
    semi = (semi0, semi1)
    semr = (semr0, semr1)

    def cid_of(t):
        return wid + t * NW

    def start_loads(t, b):
        cid = cid_of(t)

        @pl.when(cid < S_NCH)
        def _():
            base = cid * SCH
            pltpu.async_copy(dst_hbm.at[pl.ds(base, SCH)], idx[b], semi[b])
            pltpu.async_copy(msg_hbm.at[pl.ds(base, SCH)], stag[b], semr[b])

    def trip(t, b):
        cid = cid_of(t)

        @pl.when(cid < S_NCH)
        def _():
            base = cid * SCH
            pltpu.make_async_copy(dst_hbm.at[pl.ds(base, SCH)],
                                  idx[b], semi[b]).wait()
            pltpu.make_async_copy(msg_hbm.at[pl.ds(base, SCH)],
                                  stag[b], semr[b]).wait()

        start_loads(t + 1, 1 - b)

        @pl.when(cid < S_NCH)
        def _():
            pltpu.async_copy(stag[b], acc_sh.at[idx[b]], sems,
                             add=True).wait()

    start_loads(0, 0)
    pltpu.sync_copy(zeros_hbm.at[pl.ds(s * STRIPE, STRIPE)],
                    acc_sh.at[pl.ds(s * STRIPE, STRIPE)])
    plsc.subcore_barrier()

    def pair(u, _):
        trip(2 * u, 0)
        trip(2 * u + 1, 1)
        return _

    lax.fori_loop(0, S_TRIPS // 2, pair, None)

    plsc.subcore_barrier()
    pltpu.sync_copy(acc_sh.at[pl.ds(s * STRIPE, STRIPE)],
                    out_hbm.at[c, pl.ds(s * STRIPE, STRIPE)])


def _sc_scatter(msg, dst, zeros_hbm):
    mesh = plsc.VectorSubcoreMesh(core_axis_name="c", subcore_axis_name="s")
    return pl.kernel(
        _scatter_body,
        out_type=jax.ShapeDtypeStruct((NC, N_PAD, IN_C), jnp.float32),
        mesh=mesh,
        compiler_params=pltpu.CompilerParams(use_tc_tiling_on_sc=True),
        scratch_types=[
            pltpu.VMEM((SCH,), jnp.int32),
            pltpu.VMEM((SCH,), jnp.int32),
            pltpu.VMEM((SCH, IN_C), jnp.float32),
            pltpu.VMEM((SCH, IN_C), jnp.float32),
            pltpu.VMEM_SHARED((N_PAD, IN_C), jnp.float32),
            pltpu.SemaphoreType.DMA,
            pltpu.SemaphoreType.DMA,
            pltpu.SemaphoreType.DMA,
            pltpu.SemaphoreType.DMA,
            pltpu.SemaphoreType.DMA,
        ],
    )(msg, dst, zeros_hbm)


BE = 16000


def _msg_body(xj_ref, at_ref, w_ref, out_ref):
    xj = xj_ref[...]
    dn = (((0,), (1,)), ((), ()))
    acc = lax.dot_general(w_ref[D_EDGE], xj, dn,
                          preferred_element_type=jnp.float32)
    for d in range(D_EDGE):
        y = lax.dot_general(w_ref[d], xj, dn,
                            preferred_element_type=jnp.float32)
        ad = jnp.broadcast_to(at_ref[d][None, :], (OUT_C, BE))
        acc = acc + ad * y
    out_ref[:, :OUT_C] = acc.T


def _tc_msg(x_j, a_t, wstk):
    return pl.pallas_call(
        _msg_body,
        grid=(EH // BE,),
        in_specs=[
            pl.BlockSpec((BE, IN_C), lambda e: (e, 0)),
            pl.BlockSpec((D_EDGE, BE), lambda e: (0, e)),
            pl.BlockSpec((D_EDGE + 1, IN_C, OUT_C), lambda e: (0, 0, 0)),
        ],
        out_specs=pl.BlockSpec((BE, IN_C), lambda e: (e, 0)),
        out_shape=jax.ShapeDtypeStruct((EH, IN_C), jnp.float32),
    )(x_j, a_t, wstk)


BN = 1000


def _combine_body(p0_ref, p1_ref, x_ref, w_ref, b_ref, out_ref):
    out_ref[...] = (p0_ref[0, :, :OUT_C] + p0_ref[1, :, :OUT_C]
                    + p1_ref[0, :, :OUT_C] + p1_ref[1, :, :OUT_C]
                    + jnp.dot(x_ref[...], w_ref[...],
                              preferred_element_type=jnp.float32)
                    + b_ref[...])


def _tc_combine(parts0, parts1, x, w_root, bias2):
    return pl.pallas_call(
        _combine_body,
        grid=(N // BN,),
        in_specs=[
            pl.BlockSpec((NC, BN, IN_C), lambda i: (0, i, 0)),
            pl.BlockSpec((NC, BN, IN_C), lambda i: (0, i, 0)),
            pl.BlockSpec((BN, IN_C), lambda i: (i, 0)),
            pl.BlockSpec((IN_C, OUT_C), lambda i: (0, 0)),
            pl.BlockSpec((1, OUT_C), lambda i: (0, 0)),
        ],
        out_specs=pl.BlockSpec((BN, OUT_C), lambda i: (i, 0)),
        out_shape=jax.ShapeDtypeStruct((N, OUT_C), jnp.float32),
    )(parts0, parts1, x, w_root, bias2)


@jax.jit
def _run(x, edge_index, edge_attr, M, b_edge, W_root, bias):
    src = edge_index[0]
    dst = edge_index[1]
    wstk = jnp.concatenate(
        [M.reshape(D_EDGE, IN_C, OUT_C),
         b_edge.reshape(1, IN_C, OUT_C)], axis=0)
    a_t = edge_attr.T
    zeros_hbm = jnp.zeros((N_PAD, IN_C), jnp.float32)

    xj0 = _sc_gather(x, src[:EH])
    xj1 = _sc_gather(x, src[EH:])
    msg0 = _tc_msg(xj0, a_t[:, :EH], wstk)
    msg1 = _tc_msg(xj1, a_t[:, EH:], wstk)
    parts0 = _sc_scatter(msg0, dst[:EH], zeros_hbm)
    parts1 = _sc_scatter(msg1, dst[EH:], zeros_hbm)
    out = _tc_combine(parts0, parts1, x, W_root, bias.reshape(1, OUT_C))
    return out


def kernel(x, edge_index, edge_attr, M, b_edge, W_root, bias):
    out = _run(x, edge_index, edge_attr, M, b_edge, W_root, bias)
    return (out, edge_index, edge_attr)


# speedup vs baseline: 4.9467x; 1.0002x over previous
import jax
import jax.numpy as jnp
from jax import lax
from jax.experimental import pallas as pl
from jax.experimental.pallas import tpu as pltpu
from jax.experimental.pallas import tpu_sc as plsc

N = 10000
E = 160000
IN_C = 128
OUT_C = 64
D_EDGE = 4

NC, NS = 2, 16
NW = NC * NS
EPW = E // NW
CH = 200
TRIPS = EPW // CH
SPLIT = 128
N_PAD = 10240
STRIPE = N_PAD // NS


EH = E // 2
SCH_G = 128
G_NCH = EH // SCH_G
G_TRIPS = (G_NCH + NW - 1) // NW


def _make_gather_body(phase):
    ebase = phase * EH

    def _gather_body(x_hbm, src_hbm, out_hbm,
                     idx0, idx1, rows0, rows1,
                     semi0, semi1, semg, semw0, semw1):
        wid = lax.axis_index("s") * NC + lax.axis_index("c")
        idx = (idx0, idx1)
        rows = (rows0, rows1)
        semi = (semi0, semi1)
        semw = (semw0, semw1)

        def cid_of(t):
            return wid + t * NW

        def start_idx(t, b):
            cid = cid_of(t)

            @pl.when(cid < G_NCH)
            def _():
                pltpu.async_copy(src_hbm.at[pl.ds(ebase + cid * SCH_G, SCH_G)],
                                 idx[b], semi[b])

        def trip(t, b):
            cid = cid_of(t)

            @pl.when(cid < G_NCH)
            def _():
                pltpu.make_async_copy(
                    src_hbm.at[pl.ds(ebase + cid * SCH_G, SCH_G)],
                    idx[b], semi[b]).wait()

            @pl.when((t >= 2) & (cid_of(t - 2) < G_NCH))
            def _():
                pltpu.make_async_copy(
                    rows[b], out_hbm.at[pl.ds(cid_of(t - 2) * SCH_G, SCH_G)],
                    semw[b]).wait()

            start_idx(t + 1, 1 - b)

            @pl.when(cid < G_NCH)
            def _():
                pltpu.async_copy(x_hbm.at[idx[b]], rows[b], semg).wait()
                pltpu.async_copy(rows[b],
                                 out_hbm.at[pl.ds(cid * SCH_G, SCH_G)],
                                 semw[b])

        start_idx(0, 0)

        def pairs(u, _):
            trip(2 * u, 0)
            trip(2 * u + 1, 1)
            return _

        lax.fori_loop(0, G_TRIPS // 2, pairs, None)

        for tl in (G_TRIPS - 2, G_TRIPS - 1):
            cid = cid_of(tl)

            @pl.when(cid < G_NCH)
            def _():
                pltpu.make_async_copy(
                    rows[tl % 2], out_hbm.at[pl.ds(cid * SCH_G, SCH_G)],
                    semw[tl % 2]).wait()

    return _gather_body


def _sc_gather(x, src, phase):
    mesh = plsc.VectorSubcoreMesh(core_axis_name="c", subcore_axis_name="s")
    return pl.kernel(
        _make_gather_body(phase),
        out_type=jax.ShapeDtypeStruct((EH, IN_C), jnp.float32),
        mesh=mesh,
        compiler_params=pltpu.CompilerParams(use_tc_tiling_on_sc=True),
        scratch_types=[
            pltpu.VMEM((SCH_G,), jnp.int32),
            pltpu.VMEM((SCH_G,), jnp.int32),
            pltpu.VMEM((SCH_G, IN_C), jnp.float32),
            pltpu.VMEM((SCH_G, IN_C), jnp.float32),
            pltpu.SemaphoreType.DMA,
            pltpu.SemaphoreType.DMA,
            pltpu.SemaphoreType.DMA,
            pltpu.SemaphoreType.DMA,
            pltpu.SemaphoreType.DMA,
        ],
    )(x, src)


SCH = 128
S_NCH = EH // SCH
S_TRIPS = (S_NCH + NW - 1) // NW


def _make_scatter_body(phase):
    ebase = phase * EH

    def _scatter_body(msg_hbm, dst_hbm, init_hbm, out_hbm,
                      idx0, idx1, stag0, stag1, acc_sh,
                      semi0, semi1, semr0, semr1, sems):
        c = lax.axis_index("c")
        s = lax.axis_index("s")
        wid = s * NC + c
        idx = (idx0, idx1)
        stag = (stag0, stag1)
        semi = (semi0, semi1)
        semr = (semr0, semr1)

        def cid_of(t):
            return wid + t * NW

        def start_loads(t, b):
            cid = cid_of(t)

            @pl.when(cid < S_NCH)
            def _():
                base = cid * SCH
                pltpu.async_copy(dst_hbm.at[pl.ds(ebase + base, SCH)],
                                 idx[b], semi[b])
                pltpu.async_copy(msg_hbm.at[pl.ds(base, SCH)],
                                 stag[b], semr[b])

        def trip(t, b):
            cid = cid_of(t)

            @pl.when(cid < S_NCH)
            def _():
                base = cid * SCH
                pltpu.make_async_copy(dst_hbm.at[pl.ds(ebase + base, SCH)],
                                      idx[b], semi[b]).wait()
                pltpu.make_async_copy(msg_hbm.at[pl.ds(base, SCH)],
                                      stag[b], semr[b]).wait()

            start_loads(t + 1, 1 - b)

            @pl.when(cid < S_NCH)
            def _():
                pltpu.async_copy(stag[b], acc_sh.at[idx[b]], sems,
                                 add=True).wait()

        start_loads(0, 0)
        if phase == 0:
            pltpu.sync_copy(init_hbm.at[pl.ds(s * STRIPE, STRIPE)],
                            acc_sh.at[pl.ds(s * STRIPE, STRIPE)])
        else:
            pltpu.sync_copy(init_hbm.at[c, pl.ds(s * STRIPE, STRIPE)],
                            acc_sh.at[pl.ds(s * STRIPE, STRIPE)])
        plsc.subcore_barrier()

        def pair(u, _):
            trip(2 * u, 0)
            trip(2 * u + 1, 1)
            return _

        lax.fori_loop(0, S_TRIPS // 2, pair, None)

        plsc.subcore_barrier()
        pltpu.sync_copy(acc_sh.at[pl.ds(s * STRIPE, STRIPE)],
                        out_hbm.at[c, pl.ds(s * STRIPE, STRIPE)])

    return _scatter_body


def _sc_scatter(msg, dst, init_hbm, phase):
    mesh = plsc.VectorSubcoreMesh(core_axis_name="c", subcore_axis_name="s")
    return pl.kernel(
        _make_scatter_body(phase),
        out_type=jax.ShapeDtypeStruct((NC, N_PAD, IN_C), jnp.float32),
        mesh=mesh,
        compiler_params=pltpu.CompilerParams(use_tc_tiling_on_sc=True),
        scratch_types=[
            pltpu.VMEM((SCH,), jnp.int32),
            pltpu.VMEM((SCH,), jnp.int32),
            pltpu.VMEM((SCH, IN_C), jnp.float32),
            pltpu.VMEM((SCH, IN_C), jnp.float32),
            pltpu.VMEM_SHARED((N_PAD, IN_C), jnp.float32),
            pltpu.SemaphoreType.DMA,
            pltpu.SemaphoreType.DMA,
            pltpu.SemaphoreType.DMA,
            pltpu.SemaphoreType.DMA,
            pltpu.SemaphoreType.DMA,
        ],
    )(msg, dst, init_hbm)


BE = 16000


def _msg_body(xj_ref, at_ref, w_ref, out_ref):
    xj = xj_ref[...]
    dn = (((0,), (1,)), ((), ()))
    acc = lax.dot_general(w_ref[D_EDGE], xj, dn,
                          preferred_element_type=jnp.float32)
    for d in range(D_EDGE):
        y = lax.dot_general(w_ref[d], xj, dn,
                            preferred_element_type=jnp.float32)
        ad = jnp.broadcast_to(at_ref[d][None, :], (OUT_C, BE))
        acc = acc + ad * y
    out_ref[:, :OUT_C] = acc.T


def _tc_msg(x_j, a_t, wstk, phase):
    boff = phase * (EH // BE)
    return pl.pallas_call(
        _msg_body,
        grid=(EH // BE,),
        in_specs=[
            pl.BlockSpec((BE, IN_C), lambda e: (e, 0)),
            pl.BlockSpec((D_EDGE, BE), lambda e: (0, e + boff)),
            pl.BlockSpec((D_EDGE + 1, IN_C, OUT_C), lambda e: (0, 0, 0)),
        ],
        out_specs=pl.BlockSpec((BE, IN_C), lambda e: (e, 0)),
        out_shape=jax.ShapeDtypeStruct((EH, IN_C), jnp.float32),
    )(x_j, a_t, wstk)


BN = 1000


def _combine_body(p_ref, x_ref, w_ref, b_ref, out_ref):
    out_ref[...] = (p_ref[0, :, :OUT_C] + p_ref[1, :, :OUT_C]
                    + jnp.dot(x_ref[...], w_ref[...],
                              preferred_element_type=jnp.float32)
                    + b_ref[...])


def _tc_combine(parts, x, w_root, bias2):
    return pl.pallas_call(
        _combine_body,
        grid=(N // BN,),
        in_specs=[
            pl.BlockSpec((NC, BN, IN_C), lambda i: (0, i, 0)),
            pl.BlockSpec((BN, IN_C), lambda i: (i, 0)),
            pl.BlockSpec((IN_C, OUT_C), lambda i: (0, 0)),
            pl.BlockSpec((1, OUT_C), lambda i: (0, 0)),
        ],
        out_specs=pl.BlockSpec((BN, OUT_C), lambda i: (i, 0)),
        out_shape=jax.ShapeDtypeStruct((N, OUT_C), jnp.float32),
    )(parts, x, w_root, bias2)


@jax.jit
def _run(x, edge_index, edge_attr, M, b_edge, W_root, bias):
    src = edge_index[0]
    dst = edge_index[1]
    wstk = jnp.concatenate(
        [M.reshape(D_EDGE, IN_C, OUT_C),
         b_edge.reshape(1, IN_C, OUT_C)], axis=0)
    a_t = edge_attr.T
    zeros_hbm = jnp.zeros((N_PAD, IN_C), jnp.float32)

    xj0 = _sc_gather(x, src, 0)
    xj1 = _sc_gather(x, src, 1)
    msg0 = _tc_msg(xj0, a_t, wstk, 0)
    msg1 = _tc_msg(xj1, a_t, wstk, 1)
    parts0 = _sc_scatter(msg0, dst, zeros_hbm, 0)
    parts1 = _sc_scatter(msg1, dst, parts0, 1)
    out = _tc_combine(parts1, x, W_root, bias.reshape(1, OUT_C))
    return out


def kernel(x, edge_index, edge_attr, M, b_edge, W_root, bias):
    out = _run(x, edge_index, edge_attr, M, b_edge, W_root, bias)
    return (out, edge_index, edge_attr)
